# baseline (speedup 1.0000x reference)
import jax
import jax.numpy as jnp
from jax import lax
from jax.experimental import pallas as pl
from jax.experimental.pallas import tpu as pltpu
from jax.experimental.pallas import tpu_sc as plsc

NUM_EMB = 1000000
DIM = 64
BATCH = 16384

_NC = 2
_NS = 16
_NW = _NC * _NS
_B_PER_W = BATCH // _NW
_NPAIR = DIM // 2
_CH = 4
_NCHUNK = _B_PER_W // _CH


def _body(wt_hbm, idx_hbm, out_hbm, idx_v, b0, b1, b2, obuf, s0, s1, s2):
    wid = lax.axis_index("s") * _NC + lax.axis_index("c")
    base = wid * _B_PER_W

    pltpu.sync_copy(idx_hbm.at[pl.ds(base, _B_PER_W)],
                    idx_v.at[pl.ds(0, _B_PER_W)])

    iota = lax.iota(jnp.int32, 16)
    mask4 = iota < _CH
    lsel = lax.bitwise_and(iota, _CH - 1)

    def fire(c, buf, sem):
        v = idx_v[pl.ds(c * _CH, 16)]
        for k in range(_CH):
            col = lax.shift_right_logical(v[k], 7) * 128
            pltpu.async_copy(
                wt_hbm.at[:, pl.ds(col, 128)], buf.at[k], sem)

    def drain(buf, sem):
        for k in range(_CH):
            pltpu.make_async_copy(
                wt_hbm.at[:, pl.ds(0, 128)], buf.at[k], sem).wait()

    def extract(c, buf):
        v = idx_v[pl.ds(c * _CH, 16)]
        lanevec = lax.bitwise_and(v, 127)

        def word_row(k2, _):
            ev = plsc.load_gather(
                buf, [lsel, jnp.full((16,), 2 * k2, jnp.int32), lanevec],
                mask=mask4)
            od = plsc.load_gather(
                buf, [lsel, jnp.full((16,), 2 * k2 + 1, jnp.int32), lanevec],
                mask=mask4)
            packed = plsc.pack(ev, od, format=plsc.PackFormat.INTERLEAVED)
            words = plsc.bitcast(packed, jnp.int32)
            plsc.store_scatter(
                obuf, [jnp.full((16,), k2, jnp.int32), c * _CH + iota],
                words, mask=mask4)
            return 0

        lax.fori_loop(0, _NPAIR, word_row, 0)

    fire(0, b0, s0)
    fire(1, b1, s1)
    fire(2, b2, s2)

    def step(i, _):
        for j, (b, s) in enumerate(((b0, s0), (b1, s1), (b2, s2))):
            c = 3 * i + j
            drain(b, s)
            extract(c, b)

            @pl.when(c + 3 < _NCHUNK)
            def _():
                fire(c + 3, b, s)

        return 0

    lax.fori_loop(0, _NCHUNK // 3, step, 0)
    drain(b0, s0)
    extract(_NCHUNK - 2, b0)
    drain(b1, s1)
    extract(_NCHUNK - 1, b1)

    pltpu.sync_copy(obuf, out_hbm.at[:, pl.ds(base, _B_PER_W)])


_sc_gather_cast = pl.kernel(
    _body,
    out_type=jax.ShapeDtypeStruct((_NPAIR, BATCH), jnp.int32),
    mesh=plsc.VectorSubcoreMesh(
        core_axis_name="c", subcore_axis_name="s",
        num_cores=_NC, num_subcores=_NS),
    compiler_params=pltpu.CompilerParams(needs_layout_passes=False),
    scratch_types=[
        pltpu.VMEM((_B_PER_W + 16,), jnp.int32),
        pltpu.VMEM((_CH, DIM, 128), jnp.float32),
        pltpu.VMEM((_CH, DIM, 128), jnp.float32),
        pltpu.VMEM((_CH, DIM, 128), jnp.float32),
        pltpu.VMEM((_NPAIR, _B_PER_W), jnp.int32),
        pltpu.SemaphoreType.DMA,
        pltpu.SemaphoreType.DMA,
        pltpu.SemaphoreType.DMA,
    ],
)


def kernel(weights, inputs, train):
    del train
    wt = weights.T
    raw = _sc_gather_cast(wt, inputs)
    pairs = jax.lax.bitcast_convert_type(raw.T, jnp.bfloat16)
    return pairs.reshape(BATCH, DIM)


# speedup vs baseline: 1.0051x; 1.0051x over previous
import jax
import jax.numpy as jnp
from jax import lax
from jax.experimental import pallas as pl
from jax.experimental.pallas import tpu as pltpu
from jax.experimental.pallas import tpu_sc as plsc

NUM_EMB = 1000000
DIM = 64
BATCH = 16384

_NC = 2
_NS = 16
_NW = _NC * _NS
_B_PER_W = BATCH // _NW
_NPAIR = DIM // 2
_CH = 4
_NCHUNK = _B_PER_W // _CH


def _body(wt_hbm, idx_hbm, out_hbm, idx_v, b0, b1, b2, obuf, s0, s1, s2):
    wid = lax.axis_index("s") * _NC + lax.axis_index("c")
    base = wid * _B_PER_W

    pltpu.sync_copy(idx_hbm.at[pl.ds(base, _B_PER_W)],
                    idx_v.at[pl.ds(0, _B_PER_W)])

    iota = lax.iota(jnp.int32, 16)
    mask4 = iota < _CH
    lsel = lax.bitwise_and(iota, _CH - 1)

    def fire(c, buf, sem):
        v = idx_v[pl.ds(c * _CH, 16)]
        for k in range(_CH):
            col = lax.shift_right_logical(v[k], 7) * 128
            for r in range(0, DIM, 8):
                pltpu.async_copy(
                    wt_hbm.at[pl.ds(r, 8), pl.ds(col, 128)],
                    buf.at[k, pl.ds(r, 8)], sem)

    def drain(buf, sem):
        for k in range(_CH):
            pltpu.make_async_copy(
                wt_hbm.at[:, pl.ds(0, 128)], buf.at[k], sem).wait()

    def extract(c, buf):
        v = idx_v[pl.ds(c * _CH, 16)]
        lanevec = lax.bitwise_and(v, 127)

        def word_row(k2, _):
            ev = plsc.load_gather(
                buf, [lsel, jnp.full((16,), 2 * k2, jnp.int32), lanevec],
                mask=mask4)
            od = plsc.load_gather(
                buf, [lsel, jnp.full((16,), 2 * k2 + 1, jnp.int32), lanevec],
                mask=mask4)
            packed = plsc.pack(ev, od, format=plsc.PackFormat.INTERLEAVED)
            words = plsc.bitcast(packed, jnp.int32)
            plsc.store_scatter(
                obuf, [jnp.full((16,), k2, jnp.int32), c * _CH + iota],
                words, mask=mask4)
            return 0

        lax.fori_loop(0, _NPAIR, word_row, 0)

    fire(0, b0, s0)
    fire(1, b1, s1)
    fire(2, b2, s2)

    def step(i, _):
        for j, (b, s) in enumerate(((b0, s0), (b1, s1), (b2, s2))):
            c = 3 * i + j
            drain(b, s)
            extract(c, b)

            @pl.when(c + 3 < _NCHUNK)
            def _():
                fire(c + 3, b, s)

        return 0

    lax.fori_loop(0, _NCHUNK // 3, step, 0)
    drain(b0, s0)
    extract(_NCHUNK - 2, b0)
    drain(b1, s1)
    extract(_NCHUNK - 1, b1)

    pltpu.sync_copy(obuf, out_hbm.at[:, pl.ds(base, _B_PER_W)])


_sc_gather_cast = pl.kernel(
    _body,
    out_type=jax.ShapeDtypeStruct((_NPAIR, BATCH), jnp.int32),
    mesh=plsc.VectorSubcoreMesh(
        core_axis_name="c", subcore_axis_name="s",
        num_cores=_NC, num_subcores=_NS),
    compiler_params=pltpu.CompilerParams(needs_layout_passes=False),
    scratch_types=[
        pltpu.VMEM((_B_PER_W + 16,), jnp.int32),
        pltpu.VMEM((_CH, DIM, 128), jnp.float32),
        pltpu.VMEM((_CH, DIM, 128), jnp.float32),
        pltpu.VMEM((_CH, DIM, 128), jnp.float32),
        pltpu.VMEM((_NPAIR, _B_PER_W), jnp.int32),
        pltpu.SemaphoreType.DMA,
        pltpu.SemaphoreType.DMA,
        pltpu.SemaphoreType.DMA,
    ],
)


def kernel(weights, inputs, train):
    del train
    wt = weights.T
    raw = _sc_gather_cast(wt, inputs)
    pairs = jax.lax.bitcast_convert_type(raw.T, jnp.bfloat16)
    return pairs.reshape(BATCH, DIM)
